# Initial kernel scaffold; baseline (speedup 1.0000x reference)
import functools

import jax
import jax.numpy as jnp
from jax import lax
from jax.experimental import pallas as pl
from jax.experimental.pallas import tpu as pltpu
from jax.experimental.pallas import tpu_sc as plsc

_D = 200
_NC = 2
_NS = 16
_NW = _NC * _NS
_CHUNK = 128
_NBUF = 2


@functools.cache
def _make_sc_gather(B: int):
    b_per_w = B // _NW
    n_chunks = b_per_w // _CHUNK
    mesh = plsc.VectorSubcoreMesh(core_axis_name="c", subcore_axis_name="s")

    @functools.partial(
        pl.kernel,
        mesh=mesh,
        out_type=jax.ShapeDtypeStruct((B, _D), jnp.float32),
        scratch_types=[
            pltpu.VMEM((n_chunks, _CHUNK), jnp.int32),
            pltpu.VMEM((_CHUNK, _D), jnp.float32),
            pltpu.VMEM((_CHUNK, _D), jnp.float32),
            pltpu.SemaphoreType.DMA,
            pltpu.SemaphoreType.DMA,
        ],
    )
    def gather_kernel(idx_hbm, table_hbm, out_hbm, idx_v, rows0, rows1,
                      sem0, sem1):
        wid = lax.axis_index("s") * _NC + lax.axis_index("c")
        base = wid * b_per_w
        rows = (rows0, rows1)
        sems = (sem0, sem1)

        pltpu.sync_copy(idx_hbm.at[wid], idx_v)

        def gather_start(c, b):
            pltpu.async_copy(table_hbm.at[idx_v.at[c]], rows[b], sems[b])

        def gather_wait(b):
            pltpu.make_async_copy(table_hbm.at[idx_v.at[0]], rows[b],
                                  sems[b]).wait()

        for b in range(_NBUF):
            gather_start(b, b)

        def group(g, carry):
            for b in range(_NBUF):
                c = g * _NBUF + b
                gather_wait(b)
                pltpu.sync_copy(
                    rows[b], out_hbm.at[pl.ds(base + c * _CHUNK, _CHUNK)])
                nxt = c + _NBUF

                @pl.when(nxt < n_chunks)
                def _():
                    gather_start(nxt, b)
            return carry

        lax.fori_loop(0, n_chunks // _NBUF, group, 0, unroll=False)

    return gather_kernel


def kernel(input, table):
    B = input.size
    idx = input.reshape(_NW, B // _NW // _CHUNK, _CHUNK)
    out = _make_sc_gather(B)(idx, table)
    return out.reshape(*input.shape, _D)


# speedup vs baseline: 1.0716x; 1.0716x over previous
import functools

import jax
import jax.numpy as jnp
from jax import lax
from jax.experimental import pallas as pl
from jax.experimental.pallas import tpu as pltpu
from jax.experimental.pallas import tpu_sc as plsc

_D = 200
_NC = 2
_NS = 16
_NW = _NC * _NS
_CHUNK = 128
_NBUF = 2


@functools.cache
def _make_sc_gather(B: int):
    b_per_w = B // _NW
    n_chunks = b_per_w // _CHUNK
    mesh = plsc.VectorSubcoreMesh(core_axis_name="c", subcore_axis_name="s")

    @functools.partial(
        pl.kernel,
        mesh=mesh,
        out_type=jax.ShapeDtypeStruct((B, _D), jnp.float32),
        compiler_params=pltpu.CompilerParams(use_tc_tiling_on_sc=False),
        scratch_types=[
            pltpu.VMEM((n_chunks, _CHUNK), jnp.int32),
            pltpu.VMEM((_CHUNK, _D), jnp.float32),
            pltpu.VMEM((_CHUNK, _D), jnp.float32),
            pltpu.SemaphoreType.DMA,
            pltpu.SemaphoreType.DMA,
        ],
    )
    def gather_kernel(idx_hbm, table_hbm, out_hbm, idx_v, rows0, rows1,
                      sem0, sem1):
        wid = lax.axis_index("s") * _NC + lax.axis_index("c")
        base = wid * b_per_w
        rows = (rows0, rows1)
        sems = (sem0, sem1)

        pltpu.sync_copy(idx_hbm.at[wid], idx_v)

        def gather_start(c, b):
            pltpu.async_copy(table_hbm.at[idx_v.at[c]], rows[b], sems[b])

        def gather_wait(b):
            pltpu.make_async_copy(table_hbm.at[idx_v.at[0]], rows[b],
                                  sems[b]).wait()

        for b in range(_NBUF):
            gather_start(b, b)

        def group(g, carry):
            for b in range(_NBUF):
                c = g * _NBUF + b
                gather_wait(b)
                pltpu.sync_copy(
                    rows[b], out_hbm.at[pl.ds(base + c * _CHUNK, _CHUNK)])
                nxt = c + _NBUF

                @pl.when(nxt < n_chunks)
                def _():
                    gather_start(nxt, b)
            return carry

        lax.fori_loop(0, n_chunks // _NBUF, group, 0, unroll=False)

    return gather_kernel


def kernel(input, table):
    B = input.size
    idx = input.reshape(_NW, B // _NW // _CHUNK, _CHUNK)
    out = _make_sc_gather(B)(idx, table)
    return out.reshape(*input.shape, _D)


# speedup vs baseline: 2.0684x; 1.9302x over previous
import functools

import jax
import jax.numpy as jnp
from jax import lax
from jax.experimental import pallas as pl
from jax.experimental.pallas import tpu as pltpu
from jax.experimental.pallas import tpu_sc as plsc

_D = 200
_D0 = 128
_D1 = 72
_S = 50
_SP = 56
_NC = 2
_NS = 16
_NW = _NC * _NS
_NBUF = 2


@functools.cache
def _make_sc_gather(nb: int):
    b_per_w = nb // _NW
    mesh = plsc.VectorSubcoreMesh(core_axis_name="c", subcore_axis_name="s")

    @functools.partial(
        pl.kernel,
        mesh=mesh,
        out_type=jax.ShapeDtypeStruct((nb, _S, _D), jnp.float32),
        scratch_types=[
            pltpu.VMEM((b_per_w * _SP,), jnp.int32),
            [pltpu.VMEM((_S, _D0), jnp.float32) for _ in range(_NBUF)],
            [pltpu.VMEM((_S, _D0), jnp.float32) for _ in range(_NBUF)],
            [pltpu.VMEM((_S, _D1), jnp.float32) for _ in range(_NBUF)],
            [pltpu.SemaphoreType.DMA for _ in range(_NBUF)],
            [pltpu.SemaphoreType.DMA for _ in range(_NBUF)],
        ],
    )
    def gather_kernel(idx_hbm, table_hbm, t1_hbm, out_hbm, idx_v,
                      bufs0, bufs1, tails, sems0, sems1):
        wid = lax.axis_index("s") * _NC + lax.axis_index("c")
        base = wid * b_per_w
        t0 = table_hbm.at[:, pl.ds(0, _D0)]

        pltpu.sync_copy(idx_hbm.at[pl.ds(base * _SP, b_per_w * _SP)], idx_v)

        def gather_start(i, k):
            ids = idx_v.at[pl.ds(i * _SP, _S)]
            pltpu.async_copy(t0.at[ids], bufs0[k], sems0[k])
            pltpu.async_copy(t1_hbm.at[ids], bufs1[k], sems1[k])

        def gather_wait(k):
            ids = idx_v.at[pl.ds(0, _S)]
            pltpu.make_async_copy(t0.at[ids], bufs0[k], sems0[k]).wait()
            pltpu.make_async_copy(t1_hbm.at[ids], bufs1[k], sems1[k]).wait()

        for k in range(_NBUF):
            gather_start(k, k)

        def group(g, carry):
            for k in range(_NBUF):
                i = g * _NBUF + k
                b = base + i
                gather_wait(k)
                pltpu.sync_copy(bufs0[k], out_hbm.at[b, :, pl.ds(0, _D0)])
                for r in range(_S):
                    for o in (0, 16, 32, 48, 56):
                        tails[k][r, pl.ds(o, 16)] = bufs1[k][r, pl.ds(o, 16)]
                pltpu.sync_copy(tails[k], out_hbm.at[b, :, pl.ds(_D0, _D1)])
                nxt = i + _NBUF

                @pl.when(nxt < b_per_w)
                def _():
                    gather_start(nxt, k)
            return carry

        lax.fori_loop(0, b_per_w // _NBUF, group, 0, unroll=False)

    return gather_kernel


def kernel(input, table):
    nb = input.shape[0]
    idx1 = jnp.pad(input, ((0, 0), (0, _SP - _S))).reshape(-1)
    t1 = jnp.pad(table[:, _D0:], ((0, 0), (0, _D0 - _D1)))
    return _make_sc_gather(nb)(idx1, table, t1)
